# Initial kernel scaffold; baseline (speedup 1.0000x reference)
import jax
import jax.numpy as jnp
from jax.experimental import pallas as pl


def kernel(position_ids, position_embeddings):
    raise NotImplementedError("write your pallas kernel here")


# speedup vs baseline: 4.4086x; 4.4086x over previous
import functools

import jax
import jax.numpy as jnp
from jax import lax
from jax.experimental import pallas as pl
from jax.experimental.pallas import tpu as pltpu
from jax.experimental.pallas import tpu_sc as plsc


def _sc_gather_fn(B, V, D, NC, NS, C):
    NW = NC * NS
    b_per_w = B // NW
    n_chunks = b_per_w // C
    mesh = plsc.VectorSubcoreMesh(core_axis_name="c", subcore_axis_name="s")

    @functools.partial(
        pl.kernel,
        out_type=jax.ShapeDtypeStruct((B, D), jnp.float32),
        mesh=mesh,
        compiler_params=pltpu.CompilerParams(use_tc_tiling_on_sc=False),
        scratch_types=[
            pltpu.VMEM((C,), jnp.int32),
            pltpu.VMEM((C,), jnp.int32),
            pltpu.VMEM((C, D), jnp.float32),
            pltpu.VMEM((C, D), jnp.float32),
            pltpu.SemaphoreType.DMA,
            pltpu.SemaphoreType.DMA,
            pltpu.SemaphoreType.DMA,
            pltpu.SemaphoreType.DMA,
        ],
    )
    def k(idx_hbm, table_hbm, out_hbm, idx0, idx1, rows0, rows1,
          g0sem, g1sem, s0sem, s1sem):
        wid = lax.axis_index("s") * NC + lax.axis_index("c")
        base = wid * b_per_w
        idx_bufs = (idx0, idx1)
        row_bufs = (rows0, rows1)
        gsems = (g0sem, g1sem)
        ssems = (s0sem, s1sem)

        @pl.loop(0, n_chunks, step=2)
        def _outer(g):
            for b in range(2):
                chunk = g + b
                off = base + chunk * C
                @pl.when(chunk >= 2)
                def _():
                    pltpu.make_async_copy(
                        row_bufs[b], out_hbm.at[pl.ds(off - 2 * C, C)],
                        ssems[b]).wait()

                pltpu.sync_copy(idx_hbm.at[pl.ds(off, C)], idx_bufs[b])
                pltpu.async_copy(
                    table_hbm.at[idx_bufs[b]], row_bufs[b], gsems[b]).wait()
                pltpu.async_copy(
                    row_bufs[b], out_hbm.at[pl.ds(off, C)], ssems[b])

        for b in range(2):
            off = base + (n_chunks - 2 + b) * C
            pltpu.make_async_copy(
                row_bufs[b], out_hbm.at[pl.ds(off, C)], ssems[b]).wait()

    return k


def kernel(position_ids, position_embeddings):
    batch, seq = position_ids.shape
    V, D = position_embeddings.shape
    B = batch * seq
    ids_flat = position_ids.reshape(B)
    fn = _sc_gather_fn(B, V, D, 2, 16, 800)
    out = fn(ids_flat, position_embeddings)
    return out.reshape(batch, seq, D)


# speedup vs baseline: 4.4345x; 1.0059x over previous
import functools

import jax
import jax.numpy as jnp
from jax import lax
from jax.experimental import pallas as pl
from jax.experimental.pallas import tpu as pltpu
from jax.experimental.pallas import tpu_sc as plsc


def _sc_gather_fn(B, V, D, NC, NS, C):
    NW = NC * NS
    b_per_w = B // NW
    n_chunks = b_per_w // C
    mesh = plsc.VectorSubcoreMesh(core_axis_name="c", subcore_axis_name="s")

    @functools.partial(
        pl.kernel,
        out_type=jax.ShapeDtypeStruct((B, D), jnp.float32),
        mesh=mesh,
        compiler_params=pltpu.CompilerParams(use_tc_tiling_on_sc=False),
        scratch_types=[
            pltpu.VMEM((C,), jnp.int32),
            pltpu.VMEM((C,), jnp.int32),
            pltpu.VMEM((C, D), jnp.float32),
            pltpu.VMEM((C, D), jnp.float32),
            pltpu.SemaphoreType.DMA,
            pltpu.SemaphoreType.DMA,
            pltpu.SemaphoreType.DMA,
            pltpu.SemaphoreType.DMA,
            pltpu.SemaphoreType.DMA,
            pltpu.SemaphoreType.DMA,
        ],
    )
    def k(idx_hbm, table_hbm, out_hbm, idx0, idx1, rows0, rows1,
          g0sem, g1sem, s0sem, s1sem, i0sem, i1sem):
        wid = lax.axis_index("s") * NC + lax.axis_index("c")
        base = wid * b_per_w
        idx_bufs = (idx0, idx1)
        row_bufs = (rows0, rows1)
        gsems = (g0sem, g1sem)
        ssems = (s0sem, s1sem)
        isems = (i0sem, i1sem)

        def idx_slice(chunk):
            return idx_hbm.at[pl.ds(base + chunk * C, C)]

        def out_slice(chunk):
            return out_hbm.at[pl.ds(base + chunk * C, C)]

        pltpu.sync_copy(idx_slice(0), idx0)
        pltpu.async_copy(table_hbm.at[idx0], rows0, g0sem)
        pltpu.async_copy(idx_slice(1), idx1, i1sem)

        @pl.loop(0, n_chunks, step=2)
        def _outer(g):
            for b in range(2):
                o = 1 - b
                chunk = g + b

                @pl.when(chunk + 1 < n_chunks)
                def _():
                    pltpu.make_async_copy(
                        idx_slice(chunk + 1), idx_bufs[o], isems[o]).wait()

                    @pl.when(chunk >= 1)
                    def _():
                        pltpu.make_async_copy(
                            row_bufs[o], out_slice(chunk - 1),
                            ssems[o]).wait()

                    pltpu.async_copy(
                        table_hbm.at[idx_bufs[o]], row_bufs[o], gsems[o])

                pltpu.make_async_copy(
                    table_hbm.at[idx_bufs[b]], row_bufs[b], gsems[b]).wait()
                pltpu.async_copy(row_bufs[b], out_slice(chunk), ssems[b])

                @pl.when(chunk + 2 < n_chunks)
                def _():
                    pltpu.async_copy(
                        idx_slice(chunk + 2), idx_bufs[b], isems[b])

        last = n_chunks - 1
        pltpu.make_async_copy(
            row_bufs[last % 2], out_slice(last), ssems[last % 2]).wait()

    return k


def kernel(position_ids, position_embeddings):
    batch, seq = position_ids.shape
    V, D = position_embeddings.shape
    B = batch * seq
    ids_flat = position_ids.reshape(B)
    fn = _sc_gather_fn(B, V, D, 2, 16, 800)
    out = fn(ids_flat, position_embeddings)
    return out.reshape(batch, seq, D)
